# Initial kernel scaffold; baseline (speedup 1.0000x reference)
import functools

import jax
import jax.numpy as jnp
from jax import lax
from jax.experimental import pallas as pl
from jax.experimental.pallas import tpu as pltpu
from jax.experimental.pallas import tpu_sc as plsc

N = 10000
E = 320000
D = 128
NP = 10240
BM = 1024
NBLK = NP // BM
NC = 2
NS = 16
EPT = E // NS
CHUNKS = EPT // 16
RPT = NP // NS


def _tc_pre_body(h_ref, w_ref, a_ref, z_ref, hs_ref, scal_ref):
    hb = h_ref[...]
    zh = jnp.dot(hb, w_ref[0], preferred_element_type=jnp.float32)
    z = zh[:, :D]
    z_ref[0] = z
    hs_ref[0] = zh[:, D:]
    scal_ref[0] = jnp.dot(z, a_ref[0], preferred_element_type=jnp.float32)


def _tc_pre(hp, w1, a1st):
    return pl.pallas_call(
        _tc_pre_body,
        grid=(NBLK, NC),
        in_specs=[
            pl.BlockSpec((BM, D), lambda i, c: (i, 0)),
            pl.BlockSpec((1, D, 2 * D), lambda i, c: (c, 0, 0)),
            pl.BlockSpec((1, D, 2), lambda i, c: (c, 0, 0)),
        ],
        out_specs=[
            pl.BlockSpec((1, BM, D), lambda i, c: (c, i, 0)),
            pl.BlockSpec((1, BM, D), lambda i, c: (c, i, 0)),
            pl.BlockSpec((1, BM, 2), lambda i, c: (c, i, 0)),
        ],
        out_shape=[
            jax.ShapeDtypeStruct((NC, NP, D), jnp.float32),
            jax.ShapeDtypeStruct((NC, NP, D), jnp.float32),
            jax.ShapeDtypeStruct((NC, NP, 2), jnp.float32),
        ],
    )(hp, w1, a1st)


def _tc_mid_body(h_ref, hs_ref, agg_ref, es_ref, wf2t_ref, ws2t_ref, a2_ref,
                 hcat_ref, z2_ref, hs2_ref, scal2_ref):
    c = pl.program_id(1)
    hb = h_ref[...]
    es = jnp.sum(es_ref[...], axis=1)
    halves = []
    for hd in range(2):
        e = es[hd]
        denom = jnp.where(e > 0.0, e, 1.0)
        upd = hs_ref[hd] + agg_ref[hd] / denom[:, None]
        hn = jnp.where(e[:, None] > 0.0, upd, hb)
        halves.append(hb + jnp.maximum(hn, 0.0))
    hcat = jnp.concatenate(halves, axis=1)
    hcat_ref[...] = jnp.where(c == 0, halves[0], halves[1])
    z2c = jnp.dot(hcat, wf2t_ref[...], preferred_element_type=jnp.float32)
    z2_ref[0] = z2c
    hs2_ref[0] = jnp.dot(hcat, ws2t_ref[...], preferred_element_type=jnp.float32)
    part = jnp.dot(z2c, a2_ref[...], preferred_element_type=jnp.float32)

    @pl.when(c == 0)
    def _():
        scal2_ref[...] = part

    @pl.when(c != 0)
    def _():
        scal2_ref[...] = scal2_ref[...] + part


def _tc_mid(hp, hs1, agg1, esum1, wf2t, ws2t, a2st):
    return pl.pallas_call(
        _tc_mid_body,
        grid=(NBLK, NC),
        in_specs=[
            pl.BlockSpec((BM, D), lambda i, c: (i, 0)),
            pl.BlockSpec((2, BM, D), lambda i, c: (0, i, 0)),
            pl.BlockSpec((2, BM, D), lambda i, c: (0, i, 0)),
            pl.BlockSpec((2, NS, BM), lambda i, c: (0, 0, i)),
            pl.BlockSpec((2 * D, D), lambda i, c: (0, c)),
            pl.BlockSpec((2 * D, D), lambda i, c: (0, c)),
            pl.BlockSpec((D, 2), lambda i, c: (c, 0)),
        ],
        out_specs=[
            pl.BlockSpec((BM, D), lambda i, c: (i, c)),
            pl.BlockSpec((1, BM, D), lambda i, c: (c, i, 0)),
            pl.BlockSpec((1, BM, D), lambda i, c: (c, i, 0)),
            pl.BlockSpec((BM, 2), lambda i, c: (i, 0)),
        ],
        out_shape=[
            jax.ShapeDtypeStruct((NP, 2 * D), jnp.float32),
            jax.ShapeDtypeStruct((NC, NP, D), jnp.float32),
            jax.ShapeDtypeStruct((NC, NP, D), jnp.float32),
            jax.ShapeDtypeStruct((NP, 2), jnp.float32),
        ],
    )(hp, hs1, agg1, esum1, wf2t, ws2t, a2st)


def _tc_post_body(hcat_ref, hs2_ref, agg2_ref, es_ref, out_ref):
    hcb = hcat_ref[...]
    e = jnp.sum(es_ref[0], axis=0)
    denom = jnp.where(e > 0.0, e, 1.0)
    upd = hs2_ref[0] + agg2_ref[0] / denom[:, None]
    hn = jnp.where(e[:, None] > 0.0, upd, hcb)
    out_ref[...] = hcb + jnp.maximum(hn, 0.0)


def _tc_post(hcat, hs2, agg2, esum2):
    return pl.pallas_call(
        _tc_post_body,
        grid=(NBLK, NC),
        in_specs=[
            pl.BlockSpec((BM, D), lambda i, c: (i, c)),
            pl.BlockSpec((1, BM, D), lambda i, c: (c, i, 0)),
            pl.BlockSpec((1, BM, D), lambda i, c: (c, i, 0)),
            pl.BlockSpec((1, NS, BM), lambda i, c: (c, 0, i)),
        ],
        out_specs=pl.BlockSpec((BM, D), lambda i, c: (i, c)),
        out_shape=jax.ShapeDtypeStruct((NP, 2 * D), jnp.float32),
    )(hcat, hs2, agg2, esum2)


_SC_MESH = plsc.VectorSubcoreMesh(
    core_axis_name="c", subcore_axis_name="s", num_cores=NC, num_subcores=NS)


@functools.partial(
    pl.kernel,
    out_type=[
        jax.ShapeDtypeStruct((NC * NP, D), jnp.float32),
        jax.ShapeDtypeStruct((NC * NS * NP,), jnp.float32),
    ],
    mesh=_SC_MESH,
    scratch_types=[
        pltpu.VMEM((EPT,), jnp.int32),
        pltpu.VMEM((EPT,), jnp.int32),
        pltpu.VMEM((NP, 2), jnp.float32),
        pltpu.VMEM((NP,), jnp.float32),
        pltpu.VMEM((2, 16, D), jnp.float32),
        pltpu.VMEM_SHARED((NP, D), jnp.float32),
        pltpu.SemaphoreType.DMA,
    ],
)
def _sc_agg(z_hbm, asad_hbm, src_hbm, dst_hbm, agg_hbm, esum_hbm,
            srcv, dstv, asadv, esuml, rowbuf, acc, gsem):
    c = lax.axis_index("c")
    s = lax.axis_index("s")
    cn = c * NP

    pltpu.sync_copy(src_hbm.at[pl.ds(s * EPT, EPT)], srcv)
    pltpu.sync_copy(dst_hbm.at[pl.ds(s * EPT, EPT)], dstv)
    pltpu.sync_copy(asad_hbm.at[pl.ds(cn, NP)], asadv)

    zeros16 = jnp.zeros((16,), jnp.float32)

    def _z1(i, carry):
        esuml[pl.ds(i * 16, 16)] = zeros16
        return carry
    lax.fori_loop(0, NP // 16, _z1, 0)

    for j in range(16):
        for k in range(D // 16):
            rowbuf[0, j, pl.ds(k * 16, 16)] = zeros16

    def _z2(t, carry):
        pltpu.sync_copy(rowbuf.at[0], acc.at[pl.ds(s * RPT + t * 16, 16)])
        return carry
    lax.fori_loop(0, RPT // 16, _z2, 0)

    col0 = jnp.zeros((16,), jnp.int32)
    col1 = jnp.full((16,), 1, jnp.int32)

    def _issue(chunk, buf):
        idx = srcv[pl.ds(chunk * 16, 16)] + cn
        pltpu.async_copy(z_hbm.at[idx], rowbuf.at[buf], gsem)

    def _process(chunk, buf):
        src16 = srcv[pl.ds(chunk * 16, 16)]
        dst16 = dstv[pl.ds(chunk * 16, 16)]
        av = plsc.load_gather(asadv, [src16, col0])
        dv = plsc.load_gather(asadv, [dst16, col1])
        e = av + dv
        e = jnp.where(e >= 0.0, e, 0.01 * e)
        ex = jnp.exp(e)
        plsc.addupdate_scatter(esuml, [dst16], ex)
        pltpu.make_async_copy(z_hbm.at[src16 + cn], rowbuf.at[buf], gsem).wait()
        for j in range(16):
            sj = ex[j]
            for k in range(D // 16):
                sl = pl.ds(k * 16, 16)
                rowbuf[buf, j, sl] = rowbuf[buf, j, sl] * sj
        pltpu.sync_copy(rowbuf.at[buf], acc.at[dst16], add=True)

    _issue(0, 0)
    plsc.subcore_barrier()

    def _pair(i, carry):
        _issue(2 * i + 1, 1)
        _process(2 * i, 0)

        @pl.when(i < CHUNKS // 2 - 1)
        def _():
            _issue(2 * i + 2, 0)

        _process(2 * i + 1, 1)
        return carry
    lax.fori_loop(0, CHUNKS // 2, _pair, 0)

    plsc.subcore_barrier()

    pltpu.sync_copy(acc.at[pl.ds(s * RPT, RPT)],
                    agg_hbm.at[pl.ds(cn + s * RPT, RPT)])
    pltpu.sync_copy(esuml, esum_hbm.at[pl.ds((c * NS + s) * NP, NP)])


def kernel(h, e_w, snorm_n, edge_index, Ws0, Wf0, a0, Ws1, Wf1, a1,
           We, be, Ws2, Wf2, a2):
    src = edge_index[0]
    dst = edge_index[1]
    hp = jnp.pad(h, ((0, NP - N), (0, 0)))

    w1 = jnp.stack([jnp.concatenate([Wf0.T, Ws0.T], axis=1),
                    jnp.concatenate([Wf1.T, Ws1.T], axis=1)])
    a1st = jnp.stack([jnp.stack([a0[:D, 0], a0[D:, 0]], axis=1),
                      jnp.stack([a1[:D, 0], a1[D:, 0]], axis=1)])

    z1, hs1, scal1 = _tc_pre(hp, w1, a1st)
    agg1, esum1 = _sc_agg(z1.reshape(NC * NP, D),
                          scal1.reshape(NC * NP, 2), src, dst)

    a2st = jnp.stack([a2[:2 * D, 0], a2[2 * D:, 0]], axis=1)
    hcat, z2, hs2, scal2 = _tc_mid(hp, hs1, agg1.reshape(NC, NP, D),
                                   esum1.reshape(NC, NS, NP),
                                   Wf2.T, Ws2.T, a2st)

    asad2 = jnp.concatenate([scal2, scal2], axis=0)
    agg2, esum2 = _sc_agg(z2.reshape(NC * NP, D), asad2, src, dst)

    out = _tc_post(hcat, hs2, agg2.reshape(NC, NP, D),
                   esum2.reshape(NC, NS, NP))
    return out[:N]


# speedup vs baseline: 15.2123x; 15.2123x over previous
import functools

import jax
import jax.numpy as jnp
from jax import lax
from jax.experimental import pallas as pl
from jax.experimental.pallas import tpu as pltpu
from jax.experimental.pallas import tpu_sc as plsc

N = 10000
E = 320000
D = 128
NP = 10240
BM = 1024
NBLK = NP // BM
NC = 2
NS = 16
EPT = E // NS
EB = 800
NB = EPT // EB
BPAIRS = EB // 32
RPT = NP // NS


def _tc_pre_body(h_ref, w_ref, a_ref, z_ref, hs_ref, scal_ref):
    hb = h_ref[...]
    zh = jnp.dot(hb, w_ref[0], preferred_element_type=jnp.float32)
    z = zh[:, :D]
    z_ref[0] = z
    hs_ref[0] = zh[:, D:]
    ab = a_ref[0]
    scal_ref[0] = jnp.stack([
        jnp.dot(z, ab[:, 0], preferred_element_type=jnp.float32),
        jnp.dot(z, ab[:, 1], preferred_element_type=jnp.float32)])


def _tc_pre(hp, w1, a1st):
    return pl.pallas_call(
        _tc_pre_body,
        grid=(NBLK, NC),
        in_specs=[
            pl.BlockSpec((BM, D), lambda i, c: (i, 0)),
            pl.BlockSpec((1, D, 2 * D), lambda i, c: (c, 0, 0)),
            pl.BlockSpec((1, D, 2), lambda i, c: (c, 0, 0)),
        ],
        out_specs=[
            pl.BlockSpec((1, BM, D), lambda i, c: (c, i, 0)),
            pl.BlockSpec((1, BM, D), lambda i, c: (c, i, 0)),
            pl.BlockSpec((1, 2, BM), lambda i, c: (c, 0, i)),
        ],
        out_shape=[
            jax.ShapeDtypeStruct((NC, NP, D), jnp.float32),
            jax.ShapeDtypeStruct((NC, NP, D), jnp.float32),
            jax.ShapeDtypeStruct((NC, 2, NP), jnp.float32),
        ],
    )(hp, w1, a1st)


def _tc_mid_body(h_ref, hs_ref, agg_ref, es_ref, wf2t_ref, ws2t_ref, a2_ref,
                 hcat_ref, z2_ref, hs2_ref, scal2_ref):
    c = pl.program_id(1)
    hb = h_ref[...]
    es = jnp.sum(es_ref[...], axis=1)
    halves = []
    for hd in range(2):
        e = es[hd]
        denom = jnp.where(e > 0.0, e, 1.0)
        upd = hs_ref[hd] + agg_ref[hd] / denom[:, None]
        hn = jnp.where(e[:, None] > 0.0, upd, hb)
        halves.append(hb + jnp.maximum(hn, 0.0))
    hcat = jnp.concatenate(halves, axis=1)
    hcat_ref[...] = jnp.where(c == 0, halves[0], halves[1])
    z2c = jnp.dot(hcat, wf2t_ref[...], preferred_element_type=jnp.float32)
    z2_ref[0] = z2c
    hs2_ref[0] = jnp.dot(hcat, ws2t_ref[...], preferred_element_type=jnp.float32)
    ab = a2_ref[...]
    part = jnp.stack([
        jnp.dot(z2c, ab[:, 0], preferred_element_type=jnp.float32),
        jnp.dot(z2c, ab[:, 1], preferred_element_type=jnp.float32)])

    @pl.when(c == 0)
    def _():
        scal2_ref[...] = part

    @pl.when(c != 0)
    def _():
        scal2_ref[...] = scal2_ref[...] + part


def _tc_mid(hp, hs1, agg1, esum1, wf2t, ws2t, a2st):
    return pl.pallas_call(
        _tc_mid_body,
        grid=(NBLK, NC),
        in_specs=[
            pl.BlockSpec((BM, D), lambda i, c: (i, 0)),
            pl.BlockSpec((2, BM, D), lambda i, c: (0, i, 0)),
            pl.BlockSpec((2, BM, D), lambda i, c: (0, i, 0)),
            pl.BlockSpec((2, NS, BM), lambda i, c: (0, 0, i)),
            pl.BlockSpec((2 * D, D), lambda i, c: (0, c)),
            pl.BlockSpec((2 * D, D), lambda i, c: (0, c)),
            pl.BlockSpec((D, 2), lambda i, c: (c, 0)),
        ],
        out_specs=[
            pl.BlockSpec((BM, D), lambda i, c: (i, c)),
            pl.BlockSpec((1, BM, D), lambda i, c: (c, i, 0)),
            pl.BlockSpec((1, BM, D), lambda i, c: (c, i, 0)),
            pl.BlockSpec((2, BM), lambda i, c: (0, i)),
        ],
        out_shape=[
            jax.ShapeDtypeStruct((NP, 2 * D), jnp.float32),
            jax.ShapeDtypeStruct((NC, NP, D), jnp.float32),
            jax.ShapeDtypeStruct((NC, NP, D), jnp.float32),
            jax.ShapeDtypeStruct((2, NP), jnp.float32),
        ],
    )(hp, hs1, agg1, esum1, wf2t, ws2t, a2st)


def _tc_post_body(hcat_ref, hs2_ref, agg2_ref, es_ref, out_ref):
    hcb = hcat_ref[...]
    e = jnp.sum(es_ref[0], axis=0)
    denom = jnp.where(e > 0.0, e, 1.0)
    upd = hs2_ref[0] + agg2_ref[0] / denom[:, None]
    hn = jnp.where(e[:, None] > 0.0, upd, hcb)
    out_ref[...] = hcb + jnp.maximum(hn, 0.0)


def _tc_post(hcat, hs2, agg2, esum2):
    return pl.pallas_call(
        _tc_post_body,
        grid=(NBLK, NC),
        in_specs=[
            pl.BlockSpec((BM, D), lambda i, c: (i, c)),
            pl.BlockSpec((1, BM, D), lambda i, c: (c, i, 0)),
            pl.BlockSpec((1, BM, D), lambda i, c: (c, i, 0)),
            pl.BlockSpec((1, NS, BM), lambda i, c: (c, 0, i)),
        ],
        out_specs=pl.BlockSpec((BM, D), lambda i, c: (i, c)),
        out_shape=jax.ShapeDtypeStruct((NP, 2 * D), jnp.float32),
    )(hcat, hs2, agg2, esum2)


def _sc_body(z_hbm, asad_hbm, src_hbm, dst_hbm, agg_hbm, esum_hbm,
             srcb0, dstb0, srcb1, dstb1, asv, adv, esuml, rowbuf, acc,
             gsem, isem):
    c = lax.axis_index("c")
    s = lax.axis_index("s")
    cn = c * NP

    pltpu.sync_copy(asad_hbm.at[pl.ds(2 * cn, NP)], asv)
    pltpu.sync_copy(asad_hbm.at[pl.ds(2 * cn + NP, NP)], adv)

    def _stage(p, sb, db):
        off = s * EPT + p * EB
        pltpu.async_copy(src_hbm.at[pl.ds(off, EB)], sb, isem)
        pltpu.async_copy(dst_hbm.at[pl.ds(off, EB)], db, isem)

    def _wait_stage(sb, db):
        pltpu.make_async_copy(src_hbm.at[pl.ds(0, EB)], sb, isem).wait()
        pltpu.make_async_copy(src_hbm.at[pl.ds(0, EB)], db, isem).wait()

    _stage(0, srcb0, dstb0)

    zeros16 = jnp.zeros((16,), jnp.float32)

    def _z1(i, carry):
        esuml[pl.ds(i * 16, 16)] = zeros16
        return carry
    lax.fori_loop(0, NP // 16, _z1, 0)

    for j in range(16):
        for k in range(D // 16):
            rowbuf[0, j, pl.ds(k * 16, 16)] = zeros16

    def _z2(t, carry):
        pltpu.sync_copy(rowbuf.at[0], acc.at[pl.ds(s * RPT + t * 16, 16)])
        return carry
    lax.fori_loop(0, RPT // 16, _z2, 0)

    plsc.subcore_barrier()

    def _run_block(sb, db):
        def _issue(chunk, buf):
            idx = sb[pl.ds(chunk * 16, 16)] + cn
            pltpu.async_copy(z_hbm.at[idx], rowbuf.at[buf], gsem)

        def _process(chunk, buf):
            src16 = sb[pl.ds(chunk * 16, 16)]
            dst16 = db[pl.ds(chunk * 16, 16)]
            av = plsc.load_gather(asv, [src16])
            dv = plsc.load_gather(adv, [dst16])
            e = av + dv
            e = jnp.where(e >= 0.0, e, 0.01 * e)
            ex = jnp.exp(e)
            plsc.addupdate_scatter(esuml, [dst16], ex)
            pltpu.make_async_copy(
                z_hbm.at[src16 + cn], rowbuf.at[buf], gsem).wait()
            for j in range(16):
                sj = ex[j]
                for k in range(D // 16):
                    sl = pl.ds(k * 16, 16)
                    rowbuf[buf, j, sl] = rowbuf[buf, j, sl] * sj
            pltpu.sync_copy(rowbuf.at[buf], acc.at[dst16], add=True)

        _issue(0, 0)

        def _pair(i, carry):
            _issue(2 * i + 1, 1)
            _process(2 * i, 0)

            @pl.when(i < BPAIRS - 1)
            def _():
                _issue(2 * i + 2, 0)

            _process(2 * i + 1, 1)
            return carry
        lax.fori_loop(0, BPAIRS, _pair, 0)

    def _outer(p, carry):
        even = (p % 2) == 0

        @pl.when(even)
        def _():
            _wait_stage(srcb0, dstb0)

            @pl.when(p + 1 < NB)
            def _():
                _stage(p + 1, srcb1, dstb1)

            _run_block(srcb0, dstb0)

        @pl.when(jnp.logical_not(even))
        def _():
            _wait_stage(srcb1, dstb1)

            @pl.when(p + 1 < NB)
            def _():
                _stage(p + 1, srcb0, dstb0)

            _run_block(srcb1, dstb1)
        return carry
    lax.fori_loop(0, NB, _outer, 0)

    plsc.subcore_barrier()

    pltpu.sync_copy(acc.at[pl.ds(s * RPT, RPT)],
                    agg_hbm.at[pl.ds(cn + s * RPT, RPT)])
    pltpu.sync_copy(esuml, esum_hbm.at[pl.ds((c * NS + s) * NP, NP)])


@functools.cache
def _sc_agg_fn():
    mesh = plsc.VectorSubcoreMesh(
        core_axis_name="c", subcore_axis_name="s",
        num_cores=NC, num_subcores=NS)
    return pl.kernel(
        _sc_body,
        out_type=[
            jax.ShapeDtypeStruct((NC * NP, D), jnp.float32),
            jax.ShapeDtypeStruct((NC * NS * NP,), jnp.float32),
        ],
        mesh=mesh,
        compiler_params=pltpu.CompilerParams(needs_layout_passes=False),
        scratch_types=[
            pltpu.VMEM((EB,), jnp.int32),
            pltpu.VMEM((EB,), jnp.int32),
            pltpu.VMEM((EB,), jnp.int32),
            pltpu.VMEM((EB,), jnp.int32),
            pltpu.VMEM((NP,), jnp.float32),
            pltpu.VMEM((NP,), jnp.float32),
            pltpu.VMEM((NP,), jnp.float32),
            pltpu.VMEM((2, 16, D), jnp.float32),
            pltpu.VMEM_SHARED((NP, D), jnp.float32),
            pltpu.SemaphoreType.DMA,
            pltpu.SemaphoreType.DMA,
        ],
    )


def _sc_agg(z2d, asad2d, src, dst):
    return _sc_agg_fn()(z2d, asad2d, src, dst)


def kernel(h, e_w, snorm_n, edge_index, Ws0, Wf0, a0, Ws1, Wf1, a1,
           We, be, Ws2, Wf2, a2):
    src = edge_index[0]
    dst = edge_index[1]
    hp = jnp.pad(h, ((0, NP - N), (0, 0)))

    w1 = jnp.stack([jnp.concatenate([Wf0.T, Ws0.T], axis=1),
                    jnp.concatenate([Wf1.T, Ws1.T], axis=1)])
    a1st = jnp.stack([jnp.stack([a0[:D, 0], a0[D:, 0]], axis=1),
                      jnp.stack([a1[:D, 0], a1[D:, 0]], axis=1)])

    z1, hs1, scal1 = _tc_pre(hp, w1, a1st)
    agg1, esum1 = _sc_agg(z1.reshape(NC * NP, D),
                          scal1.reshape(NC * 2 * NP), src, dst)

    a2st = jnp.stack([a2[:2 * D, 0], a2[2 * D:, 0]], axis=1)
    hcat, z2, hs2, scal2 = _tc_mid(hp, hs1, agg1.reshape(NC, NP, D),
                                   esum1.reshape(NC, NS, NP),
                                   Wf2.T, Ws2.T, a2st)

    s2 = scal2.reshape(2 * NP)
    asad2 = jnp.concatenate([s2, s2], axis=0)
    agg2, esum2 = _sc_agg(z2.reshape(NC * NP, D), asad2, src, dst)

    out = _tc_post(hcat, hs2, agg2.reshape(NC, NP, D),
                   esum2.reshape(NC, NS, NP))
    return out[:N]
